# Initial kernel scaffold; baseline (speedup 1.0000x reference)
import jax
import jax.numpy as jnp
from jax.experimental import pallas as pl


def kernel(x0, x2, x3, emb_word, emb_bigram, emb_trigram, W1, b1, W2, b2):
    raise NotImplementedError("write your pallas kernel here")


# speedup vs baseline: 6.8715x; 6.8715x over previous
import functools

import jax
import jax.numpy as jnp
from jax import lax
from jax.experimental import pallas as pl
from jax.experimental.pallas import tpu as pltpu
from jax.experimental.pallas import tpu_sc as plsc

B = 4096
L = 50
D = 64
H = 256
C = 10

NC = 2
NS = 16
NW = NC * NS
ROWS_PER_W = B // NW
ROWS_PER_CHUNK = 2
IDX_PER_CHUNK = ROWS_PER_CHUNK * L
CPW = ROWS_PER_W // ROWS_PER_CHUNK
NCHUNKS = B // ROWS_PER_CHUNK
LANES = 16
G = D // LANES


def _sc_pool_body(x0_hbm, x2_hbm, x3_hbm, t0, t1, t2, out_hbm,
                  idx_v, rows_v, out_v, sem):
  c = lax.axis_index("c")
  s = lax.axis_index("s")
  wid = s * NC + c
  chunk0 = wid * CPW

  for t, (xr, tr) in enumerate(((x0_hbm, t0), (x2_hbm, t1), (x3_hbm, t2))):
    def chunk_body(i, _, xr=xr, tr=tr, t=t):
      cid = chunk0 + i
      pltpu.sync_copy(xr.at[cid], idx_v)
      pltpu.async_copy(tr.at[idx_v], rows_v, sem).wait()

      def acc_body(j, accs):
        new = []
        for r in range(ROWS_PER_CHUNK):
          for g in range(G):
            new.append(accs[r * G + g] + rows_v[r * L + j, pl.ds(g * LANES, LANES)])
        return tuple(new)

      accs = lax.fori_loop(
          0, L, acc_body,
          tuple(jnp.zeros((LANES,), jnp.float32)
                for _ in range(ROWS_PER_CHUNK * G)))
      for r in range(ROWS_PER_CHUNK):
        for g in range(G):
          out_v[i * ROWS_PER_CHUNK + r, pl.ds(t * D + g * LANES, LANES)] = (
              accs[r * G + g] * (1.0 / L))
      return 0

    lax.fori_loop(0, CPW, chunk_body, 0)

  pltpu.sync_copy(out_v, out_hbm.at[pl.ds(wid * ROWS_PER_W, ROWS_PER_W)])


_sc_pool = functools.partial(
    pl.kernel,
    out_type=jax.ShapeDtypeStruct((B, 3 * D), jnp.float32),
    mesh=plsc.VectorSubcoreMesh(
        core_axis_name="c", subcore_axis_name="s", num_cores=NC),
    scratch_types=[
        pltpu.VMEM((IDX_PER_CHUNK,), jnp.int32),
        pltpu.VMEM((IDX_PER_CHUNK, D), jnp.float32),
        pltpu.VMEM((ROWS_PER_W, 3 * D), jnp.float32),
        pltpu.SemaphoreType.DMA,
    ],
    compiler_params=pltpu.CompilerParams(use_tc_tiling_on_sc=False),
)(_sc_pool_body)


def _mlp_body(x_ref, w1_ref, b1_ref, w2_ref, b2_ref, o_ref):
  x = x_ref[...]
  h = lax.dot_general(x, w1_ref[...], (((1,), (1,)), ((), ())),
                      preferred_element_type=jnp.float32)
  h = jnp.maximum(h + b1_ref[...], 0.0)
  o = lax.dot_general(h, w2_ref[...], (((1,), (1,)), ((), ())),
                      preferred_element_type=jnp.float32)
  o_ref[...] = o + b2_ref[...]


CPAD = 128
BBLK = 1024


def _mlp(pooled, W1, b1, W2p, b2p):
  return pl.pallas_call(
      _mlp_body,
      grid=(B // BBLK,),
      in_specs=[
          pl.BlockSpec((BBLK, 3 * D), lambda i: (i, 0)),
          pl.BlockSpec((H, 3 * D), lambda i: (0, 0)),
          pl.BlockSpec((1, H), lambda i: (0, 0)),
          pl.BlockSpec((CPAD, H), lambda i: (0, 0)),
          pl.BlockSpec((1, CPAD), lambda i: (0, 0)),
      ],
      out_specs=pl.BlockSpec((BBLK, CPAD), lambda i: (i, 0)),
      out_shape=jax.ShapeDtypeStruct((B, CPAD), jnp.float32),
  )(pooled, W1, b1, W2p, b2p)


def kernel(x0, x2, x3, emb_word, emb_bigram, emb_trigram, W1, b1, W2, b2):
  x0r = x0.astype(jnp.int32).reshape(NCHUNKS, IDX_PER_CHUNK)
  x2r = x2.astype(jnp.int32).reshape(NCHUNKS, IDX_PER_CHUNK)
  x3r = x3.astype(jnp.int32).reshape(NCHUNKS, IDX_PER_CHUNK)
  pooled = _sc_pool(x0r, x2r, x3r, emb_word, emb_bigram, emb_trigram)
  W2p = jnp.zeros((CPAD, H), jnp.float32).at[:C].set(W2)
  b2p = jnp.zeros((1, CPAD), jnp.float32).at[0, :C].set(b2)
  out = _mlp(pooled, W1, b1.reshape(1, H), W2p, b2p)
  return out[:, :C]


# speedup vs baseline: 12.6020x; 1.8340x over previous
import functools

import jax
import jax.numpy as jnp
from jax import lax
from jax.experimental import pallas as pl
from jax.experimental.pallas import tpu as pltpu
from jax.experimental.pallas import tpu_sc as plsc

B = 4096
L = 50
D = 64
H = 256
C = 10

NC = 2
NS = 16
NW = NC * NS
ROWS_PER_W = B // NW
ROWS_PER_CHUNK = 2
IDX_PER_CHUNK = ROWS_PER_CHUNK * L
CPW = ROWS_PER_W // ROWS_PER_CHUNK
NCHUNKS = B // ROWS_PER_CHUNK
LANES = 16
G = D // LANES


UNROLL = 2


def _sc_pool_body(x0_hbm, x2_hbm, x3_hbm, t0, t1, t2, out_hbm,
                  idx_v, rows_v, out_v,
                  sem00, sem01, sem10, sem11, sem20, sem21):
  c = lax.axis_index("c")
  s = lax.axis_index("s")
  wid = s * NC + c
  chunk0 = wid * CPW

  xs = (x0_hbm, x2_hbm, x3_hbm)
  tabs = (t0, t1, t2)
  sems = ((sem00, sem01), (sem10, sem11), (sem20, sem21))

  for t in range(3):
    pltpu.sync_copy(xs[t].at[pl.ds(wid * CPW, CPW)], idx_v.at[t])

  def start(t, i, p):
    pltpu.async_copy(tabs[t].at[idx_v.at[t, i]], rows_v.at[t, p], sems[t][p])

  def accum(t, i, p):
    def acc_body(j, accs):
      new = list(accs)
      for u in range(UNROLL):
        for r in range(ROWS_PER_CHUNK):
          for g in range(G):
            new[r * G + g] = (
                new[r * G + g]
                + rows_v[t, p, r * L + j * UNROLL + u, pl.ds(g * LANES, LANES)])
      return tuple(new)

    accs = lax.fori_loop(
        0, L // UNROLL, acc_body,
        tuple(jnp.zeros((LANES,), jnp.float32)
              for _ in range(ROWS_PER_CHUNK * G)))
    for r in range(ROWS_PER_CHUNK):
      for g in range(G):
        out_v[i * ROWS_PER_CHUNK + r, pl.ds(t * D + g * LANES, LANES)] = (
            accs[r * G + g] * (1.0 / L))

  for t in range(3):
    start(t, 0, 0)

  def step(k, _):
    c0 = 2 * k
    for t in range(3):
      start(t, c0 + 1, 1)
    for t in range(3):
      pltpu.make_async_copy(tabs[t].at[idx_v.at[t, c0]],
                            rows_v.at[t, 0], sems[t][0]).wait()
      accum(t, c0, 0)

    @pl.when(k < CPW // 2 - 1)
    def _():
      for t in range(3):
        start(t, c0 + 2, 0)

    for t in range(3):
      pltpu.make_async_copy(tabs[t].at[idx_v.at[t, c0 + 1]],
                            rows_v.at[t, 1], sems[t][1]).wait()
      accum(t, c0 + 1, 1)
    return 0

  lax.fori_loop(0, CPW // 2, step, 0)

  pltpu.sync_copy(out_v, out_hbm.at[pl.ds(wid * ROWS_PER_W, ROWS_PER_W)])


_sc_pool = functools.partial(
    pl.kernel,
    out_type=jax.ShapeDtypeStruct((B, 3 * D), jnp.float32),
    mesh=plsc.VectorSubcoreMesh(
        core_axis_name="c", subcore_axis_name="s", num_cores=NC),
    scratch_types=[
        pltpu.VMEM((3, CPW, IDX_PER_CHUNK), jnp.int32),
        pltpu.VMEM((3, 2, IDX_PER_CHUNK, D), jnp.float32),
        pltpu.VMEM((ROWS_PER_W, 3 * D), jnp.float32),
        pltpu.SemaphoreType.DMA,
        pltpu.SemaphoreType.DMA,
        pltpu.SemaphoreType.DMA,
        pltpu.SemaphoreType.DMA,
        pltpu.SemaphoreType.DMA,
        pltpu.SemaphoreType.DMA,
    ],
    compiler_params=pltpu.CompilerParams(use_tc_tiling_on_sc=False),
)(_sc_pool_body)


def _mlp_body(x_ref, w1_ref, b1_ref, w2_ref, b2_ref, o_ref):
  x = x_ref[...]
  h = lax.dot_general(x, w1_ref[...], (((1,), (1,)), ((), ())),
                      preferred_element_type=jnp.float32)
  h = jnp.maximum(h + b1_ref[...], 0.0)
  o = lax.dot_general(h, w2_ref[...], (((1,), (1,)), ((), ())),
                      preferred_element_type=jnp.float32)
  o_ref[...] = o + b2_ref[...]


CPAD = 128
BBLK = 1024


def _mlp(pooled, W1, b1, W2p, b2p):
  return pl.pallas_call(
      _mlp_body,
      grid=(B // BBLK,),
      in_specs=[
          pl.BlockSpec((BBLK, 3 * D), lambda i: (i, 0)),
          pl.BlockSpec((H, 3 * D), lambda i: (0, 0)),
          pl.BlockSpec((1, H), lambda i: (0, 0)),
          pl.BlockSpec((CPAD, H), lambda i: (0, 0)),
          pl.BlockSpec((1, CPAD), lambda i: (0, 0)),
      ],
      out_specs=pl.BlockSpec((BBLK, CPAD), lambda i: (i, 0)),
      out_shape=jax.ShapeDtypeStruct((B, CPAD), jnp.float32),
  )(pooled, W1, b1, W2p, b2p)


def kernel(x0, x2, x3, emb_word, emb_bigram, emb_trigram, W1, b1, W2, b2):
  x0r = x0.astype(jnp.int32).reshape(NCHUNKS, IDX_PER_CHUNK)
  x2r = x2.astype(jnp.int32).reshape(NCHUNKS, IDX_PER_CHUNK)
  x3r = x3.astype(jnp.int32).reshape(NCHUNKS, IDX_PER_CHUNK)
  pooled = _sc_pool(x0r, x2r, x3r, emb_word, emb_bigram, emb_trigram)
  W2p = jnp.zeros((CPAD, H), jnp.float32).at[:C].set(W2)
  b2p = jnp.zeros((1, CPAD), jnp.float32).at[0, :C].set(b2)
  out = _mlp(pooled, W1, b1.reshape(1, H), W2p, b2p)
  return out[:, :C]
